# baseline (speedup 1.0000x reference)
import functools

import jax
import jax.numpy as jnp
from jax import lax
from jax.experimental import pallas as pl
from jax.experimental.pallas import tpu as pltpu
from jax.experimental.pallas import tpu_sc as plsc

_VOCAB = 1000000
_EMBED = 32
_WORDS = 50
_BATCH = 4096

_NC, _NS = 2, 16
_NW = _NC * _NS
_BPW = _BATCH // _NW
_GL = 16
_NG = _BPW // _GL
_RPG = _GL * _WORDS
_IDXW = 100
_IPG = _RPG // _IDXW
_IPT = _NG * _IPG


def _rsqrt(x):
    i = plsc.bitcast(x, jnp.int32)
    i = jnp.int32(0x5F3759DF) - lax.shift_right_logical(i, 1)
    y = plsc.bitcast(i, jnp.float32)
    for _ in range(3):
        y = y * (1.5 - 0.5 * x * y * y)
    return y


def _tile_body(vid_hbm, table_hbm, f0_hbm, f1_hbm, fb0_hbm, fb1_hbm, out_hbm,
               idx_v, rows_v, f0_v, f1_v, fb0_v, fb1_v, o0_v, o1_v, sem):
    wid = lax.axis_index("s") * _NC + lax.axis_index("c")

    pltpu.sync_copy(vid_hbm.at[pl.ds(wid * _IPT, _IPT)], idx_v)
    pltpu.sync_copy(f0_hbm, f0_v)
    pltpu.sync_copy(f1_hbm, f1_v)
    pltpu.sync_copy(fb0_hbm, fb0_v)
    pltpu.sync_copy(fb1_hbm, fb1_v)

    lanes = lax.iota(jnp.int32, _GL)
    bias0 = fb0_v[...]
    bias1 = fb1_v[...]
    row_base = lanes * _WORDS

    for g in range(_NG):
        copies = [
            pltpu.async_copy(
                table_hbm.at[idx_v.at[g * _IPG + k]],
                rows_v.at[pl.ds(k * _IDXW, _IDXW)],
                sem,
            )
            for k in range(_IPG)
        ]
        for c in copies:
            c.wait()

        def w_body(w, carry, _row_base=row_base, _b0=bias0, _b1=bias1):
            o0, o1 = carry
            rowv = _row_base + w
            cw = jnp.full((_GL,), w * _EMBED, jnp.int32)
            ss = jnp.zeros((_GL,), jnp.float32)
            a0 = jnp.zeros((_GL,), jnp.float32)
            a1 = jnp.zeros((_GL,), jnp.float32)
            for j in range(_EMBED):
                colv = jnp.full((_GL,), j, jnp.int32)
                d = plsc.load_gather(rows_v, [rowv, colv])
                cidx = cw + j
                c0 = plsc.load_gather(f0_v, [cidx])
                c1 = plsc.load_gather(f1_v, [cidx])
                ss = ss + d * d
                a0 = a0 + d * c0
                a1 = a1 + d * c1
            scale = _rsqrt(jnp.maximum(ss, 1.0))
            return o0 + scale * a0, o1 + scale * a1

        o0, o1 = lax.fori_loop(0, _WORDS, w_body, (bias0, bias1))
        o0_v[pl.ds(g * _GL, _GL)] = o0
        o1_v[pl.ds(g * _GL, _GL)] = o1

    pltpu.sync_copy(o0_v, out_hbm.at[0, pl.ds(wid * _BPW, _BPW)])
    pltpu.sync_copy(o1_v, out_hbm.at[1, pl.ds(wid * _BPW, _BPW)])


@jax.jit
def _sc_logits(vid2d, table, f0, f1, fb0, fb1):
    mesh = plsc.VectorSubcoreMesh(core_axis_name="c", subcore_axis_name="s")
    return pl.kernel(
        _tile_body,
        out_type=jax.ShapeDtypeStruct((2, _BATCH), jnp.float32),
        mesh=mesh,
        compiler_params=pltpu.CompilerParams(
            needs_layout_passes=False, use_tc_tiling_on_sc=False),
        scratch_types=[
            pltpu.VMEM((_IPT, _IDXW), jnp.int32),
            pltpu.VMEM((_RPG, _EMBED), jnp.float32),
            pltpu.VMEM((_WORDS * _EMBED,), jnp.float32),
            pltpu.VMEM((_WORDS * _EMBED,), jnp.float32),
            pltpu.VMEM((_GL,), jnp.float32),
            pltpu.VMEM((_GL,), jnp.float32),
            pltpu.VMEM((_BPW,), jnp.float32),
            pltpu.VMEM((_BPW,), jnp.float32),
            pltpu.SemaphoreType.DMA,
        ],
    )(vid2d, table, f0, f1, fb0, fb1)


def kernel(vocab_id, table, fc_w, fc_b):
    vid2d = vocab_id.reshape(_BATCH * _WORDS // _IDXW, _IDXW)
    fb0 = jnp.full((_GL,), fc_b[0], jnp.float32)
    fb1 = jnp.full((_GL,), fc_b[1], jnp.float32)
    out2 = _sc_logits(vid2d, table, fc_w[0], fc_w[1], fb0, fb1)
    return out2.T


# speedup vs baseline: 1.0390x; 1.0390x over previous
import jax
import jax.numpy as jnp
from jax import lax
from jax.experimental import pallas as pl
from jax.experimental.pallas import tpu as pltpu
from jax.experimental.pallas import tpu_sc as plsc

_VOCAB = 1000000
_EMBED = 32
_WORDS = 50
_BATCH = 4096

_NC, _NS = 2, 16
_NW = _NC * _NS
_BPW = _BATCH // _NW
_GL = 16
_PAIRB = 2 * _GL
_NP = _BPW // _PAIRB
_RPP = _PAIRB * _WORDS
_IDXW = 100
_IPP = _RPP // _IDXW
_IPT = _NP * _IPP


def _rsqrt(x):
    i = plsc.bitcast(x, jnp.int32)
    i = jnp.int32(0x5F3759DF) - lax.shift_right_logical(i, 1)
    y = plsc.bitcast(i, jnp.float32)
    for _ in range(3):
        y = y * (1.5 - 0.5 * x * y * y)
    return y


def _tile_body(vid_hbm, table_hbm, f0_hbm, f1_hbm, fb0_hbm, fb1_hbm, out_hbm,
               idx_v, rows_v, f0_v, f1_v, fb0_v, fb1_v, o0_v, o1_v, sem0, sem1):
    wid = lax.axis_index("s") * _NC + lax.axis_index("c")
    sems = (sem0, sem1)

    pltpu.sync_copy(vid_hbm.at[pl.ds(wid * _IPT, _IPT)], idx_v)
    pltpu.sync_copy(f0_hbm, f0_v)
    pltpu.sync_copy(f1_hbm, f1_v)
    pltpu.sync_copy(fb0_hbm, fb0_v)
    pltpu.sync_copy(fb1_hbm, fb1_v)

    lanes = lax.iota(jnp.int32, _GL)
    bias0 = fb0_v[...]
    bias1 = fb1_v[...]
    row_base = lanes * _WORDS

    def fire(p):
        buf = p % 2
        return [
            pltpu.async_copy(
                table_hbm.at[idx_v.at[p * _IPP + k]],
                rows_v.at[buf, pl.ds(k * _IDXW, _IDXW)],
                sems[buf],
            )
            for k in range(_IPP)
        ]

    pending = {0: fire(0)}
    for p in range(_NP):
        buf = p % 2
        for c in pending.pop(p):
            c.wait()
        if p + 1 < _NP:
            pending[p + 1] = fire(p + 1)

        def w_body(w, carry, _rb=row_base, _buf=buf):
            o0A, o1A, o0B, o1B = carry
            rowA = _rb + w
            rowB = rowA + _GL * _WORDS
            cw = jnp.full((_GL,), w * _EMBED, jnp.int32)
            z = jnp.zeros((_GL,), jnp.float32)
            ssA, a0A, a1A = z, z, z
            ssB, a0B, a1B = z, z, z
            for j in range(_EMBED):
                colv = jnp.full((_GL,), j, jnp.int32)
                cidx = cw + j
                c0 = plsc.load_gather(f0_v, [cidx])
                c1 = plsc.load_gather(f1_v, [cidx])
                dA = plsc.load_gather(rows_v.at[_buf], [rowA, colv])
                dB = plsc.load_gather(rows_v.at[_buf], [rowB, colv])
                ssA = ssA + dA * dA
                a0A = a0A + dA * c0
                a1A = a1A + dA * c1
                ssB = ssB + dB * dB
                a0B = a0B + dB * c0
                a1B = a1B + dB * c1
            sA = _rsqrt(jnp.maximum(ssA, 1.0))
            sB = _rsqrt(jnp.maximum(ssB, 1.0))
            return (o0A + sA * a0A, o1A + sA * a1A,
                    o0B + sB * a0B, o1B + sB * a1B)

        o0A, o1A, o0B, o1B = lax.fori_loop(
            0, _WORDS, w_body, (bias0, bias1, bias0, bias1))
        o0_v[pl.ds(p * _PAIRB, _GL)] = o0A
        o0_v[pl.ds(p * _PAIRB + _GL, _GL)] = o0B
        o1_v[pl.ds(p * _PAIRB, _GL)] = o1A
        o1_v[pl.ds(p * _PAIRB + _GL, _GL)] = o1B

    pltpu.sync_copy(o0_v, out_hbm.at[0, pl.ds(wid * _BPW, _BPW)])
    pltpu.sync_copy(o1_v, out_hbm.at[1, pl.ds(wid * _BPW, _BPW)])


@jax.jit
def _sc_logits(vid2d, table, f0, f1, fb0, fb1):
    mesh = plsc.VectorSubcoreMesh(core_axis_name="c", subcore_axis_name="s")
    return pl.kernel(
        _tile_body,
        out_type=jax.ShapeDtypeStruct((2, _BATCH), jnp.float32),
        mesh=mesh,
        compiler_params=pltpu.CompilerParams(
            needs_layout_passes=False, use_tc_tiling_on_sc=False),
        scratch_types=[
            pltpu.VMEM((_IPT, _IDXW), jnp.int32),
            pltpu.VMEM((2, _RPP, _EMBED), jnp.float32),
            pltpu.VMEM((_WORDS * _EMBED,), jnp.float32),
            pltpu.VMEM((_WORDS * _EMBED,), jnp.float32),
            pltpu.VMEM((_GL,), jnp.float32),
            pltpu.VMEM((_GL,), jnp.float32),
            pltpu.VMEM((_BPW,), jnp.float32),
            pltpu.VMEM((_BPW,), jnp.float32),
            pltpu.SemaphoreType.DMA,
            pltpu.SemaphoreType.DMA,
        ],
    )(vid2d, table, f0, f1, fb0, fb1)


def kernel(vocab_id, table, fc_w, fc_b):
    vid2d = vocab_id.reshape(_BATCH * _WORDS // _IDXW, _IDXW)
    fb0 = jnp.full((_GL,), fc_b[0], jnp.float32)
    fb1 = jnp.full((_GL,), fc_b[1], jnp.float32)
    out2 = _sc_logits(vid2d, table, fc_w[0], fc_w[1], fb0, fb1)
    return out2.T
